# Initial kernel scaffold; baseline (speedup 1.0000x reference)
import jax
import jax.numpy as jnp
from jax.experimental import pallas as pl


def kernel(x_user, x_item, edge_index_ui, edge_index_iu, edge_feat_ui, edge_feat_iu, p_ui, p_iu, ln_e_g, ln_e_b, W1, b1, W2, b2, ln_u_g, ln_u_b, ln_i_g, ln_i_b):
    raise NotImplementedError("write your pallas kernel here")


# speedup vs baseline: 1.3595x; 1.3595x over previous
import functools

import jax
import jax.numpy as jnp
from jax import lax
from jax.experimental import pallas as pl
from jax.experimental.pallas import tpu as pltpu
from jax.experimental.pallas import tpu_sc as plsc

N = 10000
E = 160000
D = 256
DE = 16
H = 128
DH = 128

NTILES = 16
EPT = E // NTILES
CH = 80
NCHUNK = EPT // CH
ROWS_A = 640
ROWS_B = N - 15 * ROWS_A


_EBLK = 1280


def _prompt_body(ef_ref, g_ref, b_ref, w1_ref, b1_ref, w2_ref, b2_ref, pv_ref,
                 lo_ref, hi_ref):
    ef = ef_ref[...]
    mu = jnp.mean(ef, axis=1, keepdims=True)
    xc = ef - mu
    var = jnp.mean(xc * xc, axis=1, keepdims=True)
    h = xc * lax.rsqrt(var + 1e-5) * g_ref[...] + b_ref[...]
    h1 = jnp.maximum(
        jnp.dot(h, w1_ref[...], preferred_element_type=jnp.float32) + b1_ref[...],
        0.0)
    p = pv_ref[...] + 0.5 * (
        jnp.dot(h1, w2_ref[...], preferred_element_type=jnp.float32) + b2_ref[...])
    lo_ref[...] = p[:, :DH]
    hi_ref[...] = p[:, DH:]


def _edge_prompt(ef, ln_g, ln_b, W1, b1, W2, b2, p_vec):
    grid = (E // _EBLK,)
    return pl.pallas_call(
        _prompt_body,
        grid=grid,
        in_specs=[
            pl.BlockSpec((_EBLK, DE), lambda i: (i, 0)),
            pl.BlockSpec((1, DE), lambda i: (0, 0)),
            pl.BlockSpec((1, DE), lambda i: (0, 0)),
            pl.BlockSpec((DE, H), lambda i: (0, 0)),
            pl.BlockSpec((1, H), lambda i: (0, 0)),
            pl.BlockSpec((H, D), lambda i: (0, 0)),
            pl.BlockSpec((1, D), lambda i: (0, 0)),
            pl.BlockSpec((1, D), lambda i: (0, 0)),
        ],
        out_specs=[
            pl.BlockSpec((_EBLK, DH), lambda i: (i, 0)),
            pl.BlockSpec((_EBLK, DH), lambda i: (i, 0)),
        ],
        out_shape=[
            jax.ShapeDtypeStruct((E, DH), jnp.float32),
            jax.ShapeDtypeStruct((E, DH), jnp.float32),
        ],
    )(ef, ln_g.reshape(1, DE), ln_b.reshape(1, DE), W1, b1.reshape(1, H), W2,
      b2.reshape(1, D), p_vec.reshape(1, D))


@functools.cache
def _make_sc_agg():
    mesh = plsc.VectorSubcoreMesh(core_axis_name="c", subcore_axis_name="s")

    @functools.partial(
        pl.kernel,
        mesh=mesh,
        out_type=[
            jax.ShapeDtypeStruct((2 * N, DH), jnp.float32),
            jax.ShapeDtypeStruct((N, 128), jnp.float32),
        ],
        scratch_types=[
            pltpu.VMEM((CH,), jnp.int32),
            pltpu.VMEM((CH,), jnp.int32),
            pltpu.VMEM((CH, DH), jnp.float32),
            pltpu.VMEM((CH, DH), jnp.float32),
            pltpu.VMEM_SHARED((N, DH), jnp.float32),
            pltpu.SemaphoreType.DMA,
        ],
    )
    def _sc_agg(x_cat, src_hbm, dst_hbm, p_cat,
                out_cat, deg_out,
                src_v, dst_v, x_rows, p_rows, acc, sem):
        c = lax.axis_index("c")
        s = lax.axis_index("s")
        xoff = c * N
        poff = c * E

        zv = jnp.zeros((16,), jnp.float32)

        def fill_x_rows(val):
            def body(e, carry):
                for j in range(DH // 16):
                    x_rows[e, pl.ds(j * 16, 16)] = val
                return carry
            lax.fori_loop(0, CH, body, 0)

        def zero_acc_span():
            def span(nspan):
                for j in range(nspan):
                    sl = pl.ds(s * ROWS_A + j * CH, CH)
                    pltpu.sync_copy(x_rows, acc.at[sl])

            @pl.when(s < NTILES - 1)
            def _():
                span(ROWS_A // CH)

            @pl.when(s == NTILES - 1)
            def _():
                span(ROWS_B // CH)

        fill_x_rows(zv)
        zero_acc_span()
        plsc.subcore_barrier()

        def chunk(i, carry):
            base = s * EPT + i * CH
            pltpu.sync_copy(src_hbm.at[pl.ds(poff + base, CH)], src_v)
            pltpu.sync_copy(dst_hbm.at[pl.ds(base, CH)], dst_v)
            pltpu.sync_copy(p_cat.at[pl.ds(poff + base, CH)], p_rows)
            pltpu.async_copy(x_cat.at[src_v], x_rows, sem).wait()

            def mrow(e, inner):
                for j in range(DH // 16):
                    sl = pl.ds(j * 16, 16)
                    x_rows[e, sl] = x_rows[e, sl] * p_rows[e, sl]
                return inner

            lax.fori_loop(0, CH, mrow, 0)

            pltpu.sync_copy(x_rows, acc.at[dst_v], add=True)
            return carry

        lax.fori_loop(0, NCHUNK, chunk, 0)

        plsc.subcore_barrier()

        def dump_agg(nspan):
            for j in range(nspan):
                sl = pl.ds(s * ROWS_A + j * CH, CH)
                osl = pl.ds(xoff + s * ROWS_A + j * CH, CH)
                pltpu.sync_copy(acc.at[sl], x_rows)
                pltpu.sync_copy(x_rows, out_cat.at[osl])

        @pl.when(s < NTILES - 1)
        def _():
            dump_agg(ROWS_A // CH)

        @pl.when(s == NTILES - 1)
        def _():
            dump_agg(ROWS_B // CH)

        plsc.subcore_barrier()

        fill_x_rows(zv)
        zero_acc_span()
        ov = jnp.full((16,), 1.0, jnp.float32)

        def fill_ones(e, carry):
            for j in range(DH // 16):
                p_rows[e, pl.ds(j * 16, 16)] = ov
            return carry

        lax.fori_loop(0, CH, fill_ones, 0)
        plsc.subcore_barrier()

        def chunk_deg(i, carry):
            base = s * EPT + i * CH
            pltpu.sync_copy(dst_hbm.at[pl.ds(base, CH)], dst_v)
            pltpu.sync_copy(p_rows, acc.at[dst_v], add=True)
            return carry

        lax.fori_loop(0, NCHUNK, chunk_deg, 0)

        plsc.subcore_barrier()

        HN = N // 2
        DR_A = 320
        DR_B = HN - 15 * DR_A

        def dump_deg(spans):
            for (off, ln) in spans:
                pltpu.sync_copy(acc.at[pl.ds(c * HN + off, ln)], x_rows.at[pl.ds(0, ln)])
                pltpu.sync_copy(x_rows.at[pl.ds(0, ln)],
                                deg_out.at[pl.ds(c * HN + off, ln)])

        @pl.when(s < NTILES - 1)
        def _():
            dump_deg([(s * DR_A + j * CH, CH) for j in range(DR_A // CH)])

        @pl.when(s == NTILES - 1)
        def _():
            dump_deg([(15 * DR_A, CH), (15 * DR_A + CH, CH),
                      (15 * DR_A + 2 * CH, DR_B - 2 * CH)])

    return _sc_agg


_NBLK = 1000


def _fin_body(x_ref, lo_ref, hi_ref, deg_ref, g_ref, b_ref, out_ref):
    x = x_ref[...]
    agg = jnp.concatenate([lo_ref[...], hi_ref[...]], axis=1)
    deg = jnp.maximum(deg_ref[:, 0:1], 1.0)
    h = x + 0.5 * agg / deg
    mu = jnp.mean(h, axis=1, keepdims=True)
    xc = h - mu
    var = jnp.mean(xc * xc, axis=1, keepdims=True)
    out_ref[...] = xc * lax.rsqrt(var + 1e-5) * g_ref[...] + b_ref[...]


def _finalize(x, agg_lo, agg_hi, deg, g, b):
    grid = (N // _NBLK,)
    return pl.pallas_call(
        _fin_body,
        grid=grid,
        in_specs=[
            pl.BlockSpec((_NBLK, D), lambda i: (i, 0)),
            pl.BlockSpec((_NBLK, DH), lambda i: (i, 0)),
            pl.BlockSpec((_NBLK, DH), lambda i: (i, 0)),
            pl.BlockSpec((_NBLK, 16), lambda i: (i, 0)),
            pl.BlockSpec((1, D), lambda i: (0, 0)),
            pl.BlockSpec((1, D), lambda i: (0, 0)),
        ],
        out_specs=pl.BlockSpec((_NBLK, D), lambda i: (i, 0)),
        out_shape=jax.ShapeDtypeStruct((N, D), jnp.float32),
    )(x, agg_lo, agg_hi, deg, g.reshape(1, D), b.reshape(1, D))


def kernel(x_user, x_item, edge_index_ui, edge_index_iu, edge_feat_ui,
           edge_feat_iu, p_ui, p_iu, ln_e_g, ln_e_b, W1, b1, W2, b2,
           ln_u_g, ln_u_b, ln_i_g, ln_i_b):
    src_ui, dst_ui = edge_index_ui[0], edge_index_ui[1]
    src_iu, dst_iu = edge_index_iu[0], edge_index_iu[1]

    p_ui_lo, p_ui_hi = _edge_prompt(edge_feat_ui, ln_e_g, ln_e_b, W1, b1, W2,
                                    b2, p_ui)
    p_iu_lo, p_iu_hi = _edge_prompt(edge_feat_iu, ln_e_g, ln_e_b, W1, b1, W2,
                                    b2, p_iu)

    xu_cat = jnp.concatenate([x_user[:, :DH], x_user[:, DH:]], axis=0)
    xi_cat = jnp.concatenate([x_item[:, :DH], x_item[:, DH:]], axis=0)
    p_ui_cat = jnp.concatenate([p_ui_lo, p_ui_hi], axis=0)
    p_iu_cat = jnp.concatenate([p_iu_lo, p_iu_hi], axis=0)

    src_ui2 = jnp.concatenate([src_ui, src_ui + N])
    src_iu2 = jnp.concatenate([src_iu, src_iu + N])

    sc_agg = _make_sc_agg()
    agg_i, deg_i = sc_agg(xu_cat, src_ui2, dst_ui, p_ui_cat)
    agg_u, deg_u = sc_agg(xi_cat, src_iu2, dst_iu, p_iu_cat)

    def deg16(dcounters):
        return jnp.broadcast_to(dcounters[:, 0:1], (N, 16))

    out_user = _finalize(x_user, agg_u[:N], agg_u[N:], deg16(deg_u),
                         ln_u_g, ln_u_b)
    out_item = _finalize(x_item, agg_i[:N], agg_i[N:], deg16(deg_i),
                         ln_i_g, ln_i_b)
    return (out_user, out_item)


# speedup vs baseline: 2.0284x; 1.4920x over previous
import functools

import jax
import jax.numpy as jnp
from jax import lax
from jax.experimental import pallas as pl
from jax.experimental.pallas import tpu as pltpu
from jax.experimental.pallas import tpu_sc as plsc

N = 10000
E = 160000
D = 256
DE = 16
H = 128
DH = 128

NTILES = 16
EPT = E // NTILES
CH = 80
NCHUNK = EPT // CH
ROWS_A = 640
ROWS_B = N - 15 * ROWS_A


_EBLK = 1280


def _prompt_body(ef_ref, g_ref, b_ref, w1_ref, b1_ref, w2_ref, b2_ref, pv_ref,
                 lo_ref, hi_ref):
    ef = ef_ref[...]
    mu = jnp.mean(ef, axis=1, keepdims=True)
    xc = ef - mu
    var = jnp.mean(xc * xc, axis=1, keepdims=True)
    h = xc * lax.rsqrt(var + 1e-5) * g_ref[...] + b_ref[...]
    h1 = jnp.maximum(
        jnp.dot(h, w1_ref[...], preferred_element_type=jnp.float32) + b1_ref[...],
        0.0)
    p = pv_ref[...] + 0.5 * (
        jnp.dot(h1, w2_ref[...], preferred_element_type=jnp.float32) + b2_ref[...])
    lo_ref[...] = p[:, :DH]
    hi_ref[...] = p[:, DH:]


def _edge_prompt(ef, ln_g, ln_b, W1, b1, W2, b2, p_vec):
    grid = (E // _EBLK,)
    return pl.pallas_call(
        _prompt_body,
        grid=grid,
        in_specs=[
            pl.BlockSpec((_EBLK, DE), lambda i: (i, 0)),
            pl.BlockSpec((1, DE), lambda i: (0, 0)),
            pl.BlockSpec((1, DE), lambda i: (0, 0)),
            pl.BlockSpec((DE, H), lambda i: (0, 0)),
            pl.BlockSpec((1, H), lambda i: (0, 0)),
            pl.BlockSpec((H, D), lambda i: (0, 0)),
            pl.BlockSpec((1, D), lambda i: (0, 0)),
            pl.BlockSpec((1, D), lambda i: (0, 0)),
        ],
        out_specs=[
            pl.BlockSpec((_EBLK, DH), lambda i: (i, 0)),
            pl.BlockSpec((_EBLK, DH), lambda i: (i, 0)),
        ],
        out_shape=[
            jax.ShapeDtypeStruct((E, DH), jnp.float32),
            jax.ShapeDtypeStruct((E, DH), jnp.float32),
        ],
    )(ef, ln_g.reshape(1, DE), ln_b.reshape(1, DE), W1, b1.reshape(1, H), W2,
      b2.reshape(1, D), p_vec.reshape(1, D))


@functools.cache
def _make_sc_agg():
    mesh = plsc.VectorSubcoreMesh(core_axis_name="c", subcore_axis_name="s")

    @functools.partial(
        pl.kernel,
        mesh=mesh,
        out_type=[
            jax.ShapeDtypeStruct((2 * N, DH), jnp.float32),
            jax.ShapeDtypeStruct((N, 128), jnp.float32),
        ],
        scratch_types=[
            pltpu.VMEM((CH,), jnp.int32),
            pltpu.VMEM((CH,), jnp.int32),
            pltpu.VMEM((CH,), jnp.int32),
            pltpu.VMEM((CH,), jnp.int32),
            pltpu.VMEM((CH, DH), jnp.float32),
            pltpu.VMEM((CH, DH), jnp.float32),
            pltpu.VMEM((CH, DH), jnp.float32),
            pltpu.VMEM((CH, DH), jnp.float32),
            pltpu.VMEM_SHARED((N, DH), jnp.float32),
            pltpu.SemaphoreType.DMA,
            pltpu.SemaphoreType.DMA,
            pltpu.SemaphoreType.DMA,
            pltpu.SemaphoreType.DMA,
        ],
    )
    def _sc_agg(x_cat, src_hbm, dst_hbm, p_cat,
                out_cat, deg_out,
                src_v0, src_v1, dst_v0, dst_v1, xr0, xr1, pr0, pr1,
                acc, semL0, semL1, semG0, semG1):
        c = lax.axis_index("c")
        s = lax.axis_index("s")
        xoff = c * N
        poff = c * E

        B = [(src_v0, dst_v0, xr0, pr0, semL0, semG0),
             (src_v1, dst_v1, xr1, pr1, semL1, semG1)]

        zv = jnp.zeros((16,), jnp.float32)

        def fill_rows(ref, val):
            def body(e, carry):
                for j in range(DH // 16):
                    ref[e, pl.ds(j * 16, 16)] = val
                return carry
            lax.fori_loop(0, CH, body, 0)

        def zero_acc_span():
            def span(nspan):
                for j in range(nspan):
                    sl = pl.ds(s * ROWS_A + j * CH, CH)
                    pltpu.sync_copy(xr0, acc.at[sl])

            @pl.when(s < NTILES - 1)
            def _():
                span(ROWS_A // CH)

            @pl.when(s == NTILES - 1)
            def _():
                span(ROWS_B // CH)

        def issue_loads(ch, b):
            src_v, dst_v, _, p_rows, semL, _ = B[b]
            base = s * EPT + ch * CH
            pltpu.async_copy(src_hbm.at[pl.ds(poff + base, CH)], src_v, semL)
            pltpu.async_copy(dst_hbm.at[pl.ds(base, CH)], dst_v, semL)
            pltpu.async_copy(p_cat.at[pl.ds(poff + base, CH)], p_rows, semL)

        def drain_loads(b):
            src_v, dst_v, _, p_rows, semL, _ = B[b]
            pltpu.make_async_copy(src_hbm.at[pl.ds(0, CH)], src_v, semL).wait()
            pltpu.make_async_copy(dst_hbm.at[pl.ds(0, CH)], dst_v, semL).wait()
            pltpu.make_async_copy(p_cat.at[pl.ds(0, CH)], p_rows, semL).wait()

        def issue_gather(b):
            src_v, _, x_rows, _, _, semG = B[b]
            pltpu.async_copy(x_cat.at[src_v], x_rows, semG)

        def drain_gather(b):
            _, _, x_rows, _, _, semG = B[b]
            pltpu.make_async_copy(x_cat.at[pl.ds(0, CH)], x_rows, semG).wait()

        def mul_scatter(b):
            _, dst_v, x_rows, p_rows, _, _ = B[b]

            def mrow(e, inner):
                for j in range(DH // 16):
                    sl = pl.ds(j * 16, 16)
                    x_rows[e, sl] = x_rows[e, sl] * p_rows[e, sl]
                return inner

            lax.fori_loop(0, CH, mrow, 0)
            pltpu.sync_copy(x_rows, acc.at[dst_v], add=True)

        fill_rows(xr0, zv)
        zero_acc_span()
        plsc.subcore_barrier()

        issue_loads(0, 0)

        def pair(m, carry):
            for b in (0, 1):
                ch = 2 * m + b

                @pl.when(ch < NCHUNK)
                def _():
                    drain_loads(b)
                    issue_gather(b)

                @pl.when(ch >= 1)
                def _():
                    drain_gather(1 - b)
                    mul_scatter(1 - b)

                @pl.when(ch < NCHUNK - 1)
                def _():
                    issue_loads(ch + 1, 1 - b)
            return carry

        lax.fori_loop(0, (NCHUNK + 2) // 2, pair, 0)

        plsc.subcore_barrier()

        def dump_agg(nspan):
            for j in range(nspan):
                sl = pl.ds(s * ROWS_A + j * CH, CH)
                osl = pl.ds(xoff + s * ROWS_A + j * CH, CH)
                pltpu.sync_copy(acc.at[sl], xr0)
                pltpu.sync_copy(xr0, out_cat.at[osl])

        @pl.when(s < NTILES - 1)
        def _():
            dump_agg(ROWS_A // CH)

        @pl.when(s == NTILES - 1)
        def _():
            dump_agg(ROWS_B // CH)

        plsc.subcore_barrier()

        fill_rows(xr0, zv)
        zero_acc_span()
        ov = jnp.full((16,), 1.0, jnp.float32)
        fill_rows(pr0, ov)
        plsc.subcore_barrier()

        def issue_dst(ch, b):
            dst_v, semL = B[b][1], B[b][4]
            pltpu.async_copy(dst_hbm.at[pl.ds(s * EPT + ch * CH, CH)],
                             dst_v, semL)

        def drain_dst(b):
            dst_v, semL = B[b][1], B[b][4]
            pltpu.make_async_copy(dst_hbm.at[pl.ds(0, CH)], dst_v, semL).wait()

        issue_dst(0, 0)

        def pair_deg(m, carry):
            for b in (0, 1):
                ch = 2 * m + b

                @pl.when(ch < NCHUNK)
                def _():
                    drain_dst(b)

                @pl.when(ch < NCHUNK - 1)
                def _():
                    issue_dst(ch + 1, 1 - b)

                @pl.when(ch < NCHUNK)
                def _():
                    pltpu.sync_copy(pr0, acc.at[B[b][1]], add=True)
            return carry

        lax.fori_loop(0, (NCHUNK + 1) // 2, pair_deg, 0)

        plsc.subcore_barrier()

        HN = N // 2
        DR_A = 320
        DR_B = HN - 15 * DR_A

        def dump_deg(spans):
            for (off, ln) in spans:
                pltpu.sync_copy(acc.at[pl.ds(c * HN + off, ln)],
                                xr0.at[pl.ds(0, ln)])
                pltpu.sync_copy(xr0.at[pl.ds(0, ln)],
                                deg_out.at[pl.ds(c * HN + off, ln)])

        @pl.when(s < NTILES - 1)
        def _():
            dump_deg([(s * DR_A + j * CH, CH) for j in range(DR_A // CH)])

        @pl.when(s == NTILES - 1)
        def _():
            dump_deg([(15 * DR_A, CH), (15 * DR_A + CH, CH),
                      (15 * DR_A + 2 * CH, DR_B - 2 * CH)])

    return _sc_agg


_NBLK = 1000


def _fin_body(x_ref, lo_ref, hi_ref, deg_ref, g_ref, b_ref, out_ref):
    x = x_ref[...]
    agg = jnp.concatenate([lo_ref[...], hi_ref[...]], axis=1)
    deg = jnp.maximum(deg_ref[:, 0:1], 1.0)
    h = x + 0.5 * agg / deg
    mu = jnp.mean(h, axis=1, keepdims=True)
    xc = h - mu
    var = jnp.mean(xc * xc, axis=1, keepdims=True)
    out_ref[...] = xc * lax.rsqrt(var + 1e-5) * g_ref[...] + b_ref[...]


def _finalize(x, agg_lo, agg_hi, deg, g, b):
    grid = (N // _NBLK,)
    return pl.pallas_call(
        _fin_body,
        grid=grid,
        in_specs=[
            pl.BlockSpec((_NBLK, D), lambda i: (i, 0)),
            pl.BlockSpec((_NBLK, DH), lambda i: (i, 0)),
            pl.BlockSpec((_NBLK, DH), lambda i: (i, 0)),
            pl.BlockSpec((_NBLK, 16), lambda i: (i, 0)),
            pl.BlockSpec((1, D), lambda i: (0, 0)),
            pl.BlockSpec((1, D), lambda i: (0, 0)),
        ],
        out_specs=pl.BlockSpec((_NBLK, D), lambda i: (i, 0)),
        out_shape=jax.ShapeDtypeStruct((N, D), jnp.float32),
    )(x, agg_lo, agg_hi, deg, g.reshape(1, D), b.reshape(1, D))


def kernel(x_user, x_item, edge_index_ui, edge_index_iu, edge_feat_ui,
           edge_feat_iu, p_ui, p_iu, ln_e_g, ln_e_b, W1, b1, W2, b2,
           ln_u_g, ln_u_b, ln_i_g, ln_i_b):
    src_ui, dst_ui = edge_index_ui[0], edge_index_ui[1]
    src_iu, dst_iu = edge_index_iu[0], edge_index_iu[1]

    p_ui_lo, p_ui_hi = _edge_prompt(edge_feat_ui, ln_e_g, ln_e_b, W1, b1, W2,
                                    b2, p_ui)
    p_iu_lo, p_iu_hi = _edge_prompt(edge_feat_iu, ln_e_g, ln_e_b, W1, b1, W2,
                                    b2, p_iu)

    xu_cat = jnp.concatenate([x_user[:, :DH], x_user[:, DH:]], axis=0)
    xi_cat = jnp.concatenate([x_item[:, :DH], x_item[:, DH:]], axis=0)
    p_ui_cat = jnp.concatenate([p_ui_lo, p_ui_hi], axis=0)
    p_iu_cat = jnp.concatenate([p_iu_lo, p_iu_hi], axis=0)

    src_ui2 = jnp.concatenate([src_ui, src_ui + N])
    src_iu2 = jnp.concatenate([src_iu, src_iu + N])

    sc_agg = _make_sc_agg()
    agg_i, deg_i = sc_agg(xu_cat, src_ui2, dst_ui, p_ui_cat)
    agg_u, deg_u = sc_agg(xi_cat, src_iu2, dst_iu, p_iu_cat)

    def deg16(dcounters):
        return jnp.broadcast_to(dcounters[:, 0:1], (N, 16))

    out_user = _finalize(x_user, agg_u[:N], agg_u[N:], deg16(deg_u),
                         ln_u_g, ln_u_b)
    out_item = _finalize(x_item, agg_i[:N], agg_i[N:], deg16(deg_i),
                         ln_i_g, ln_i_b)
    return (out_user, out_item)


# speedup vs baseline: 2.4278x; 1.1969x over previous
import functools

import jax
import jax.numpy as jnp
from jax import lax
from jax.experimental import pallas as pl
from jax.experimental.pallas import tpu as pltpu
from jax.experimental.pallas import tpu_sc as plsc

N = 10000
E = 160000
D = 256
DE = 16
H = 128
DH = 128

NTILES = 16
EPT = E // NTILES
CH = 80
NCHUNK = EPT // CH
ROWS_A = 640
ROWS_B = N - 15 * ROWS_A


_EBLK = 1280


def _prompt_body(ef_ref, g_ref, b_ref, w1_ref, b1_ref, w2_ref, b2_ref, pv_ref,
                 lo_ref):
    ef = ef_ref[...]
    mu = jnp.mean(ef, axis=1, keepdims=True)
    xc = ef - mu
    var = jnp.mean(xc * xc, axis=1, keepdims=True)
    h = xc * lax.rsqrt(var + 1e-5) * g_ref[...] + b_ref[...]
    h1 = jnp.maximum(
        jnp.dot(h, w1_ref[...], preferred_element_type=jnp.float32) + b1_ref[...],
        0.0)
    p = pv_ref[...] + 0.5 * (
        jnp.dot(h1, w2_ref[...], preferred_element_type=jnp.float32) + b2_ref[...])
    lo_ref[0] = p[:, :DH]
    lo_ref[1] = p[:, DH:]


def _edge_prompt(ef, ln_g, ln_b, W1, b1, W2, b2, p_vec):
    grid = (E // _EBLK,)
    return pl.pallas_call(
        _prompt_body,
        grid=grid,
        in_specs=[
            pl.BlockSpec((_EBLK, DE), lambda i: (i, 0)),
            pl.BlockSpec((1, DE), lambda i: (0, 0)),
            pl.BlockSpec((1, DE), lambda i: (0, 0)),
            pl.BlockSpec((DE, H), lambda i: (0, 0)),
            pl.BlockSpec((1, H), lambda i: (0, 0)),
            pl.BlockSpec((H, D), lambda i: (0, 0)),
            pl.BlockSpec((1, D), lambda i: (0, 0)),
            pl.BlockSpec((1, D), lambda i: (0, 0)),
        ],
        out_specs=pl.BlockSpec((2, _EBLK, DH), lambda i: (0, i, 0)),
        out_shape=jax.ShapeDtypeStruct((2, E, DH), jnp.float32),
    )(ef, ln_g.reshape(1, DE), ln_b.reshape(1, DE), W1, b1.reshape(1, H), W2,
      b2.reshape(1, D), p_vec.reshape(1, D))


@functools.cache
def _make_sc_agg():
    mesh = plsc.VectorSubcoreMesh(core_axis_name="c", subcore_axis_name="s")

    @functools.partial(
        pl.kernel,
        mesh=mesh,
        out_type=[
            jax.ShapeDtypeStruct((2 * N, DH), jnp.float32),
            jax.ShapeDtypeStruct((N, 128), jnp.float32),
        ],
        scratch_types=[
            pltpu.VMEM((CH,), jnp.int32),
            pltpu.VMEM((CH,), jnp.int32),
            pltpu.VMEM((CH,), jnp.int32),
            pltpu.VMEM((CH,), jnp.int32),
            pltpu.VMEM((CH, DH), jnp.float32),
            pltpu.VMEM((CH, DH), jnp.float32),
            pltpu.VMEM((CH, DH), jnp.float32),
            pltpu.VMEM((CH, DH), jnp.float32),
            pltpu.VMEM_SHARED((N, DH), jnp.float32),
            pltpu.SemaphoreType.DMA,
            pltpu.SemaphoreType.DMA,
            pltpu.SemaphoreType.DMA,
            pltpu.SemaphoreType.DMA,
        ],
    )
    def _sc_agg(x_cat, src_hbm, dst_hbm, p_cat,
                out_cat, deg_out,
                src_v0, src_v1, dst_v0, dst_v1, xr0, xr1, pr0, pr1,
                acc, semL0, semL1, semG0, semG1):
        c = lax.axis_index("c")
        s = lax.axis_index("s")
        xoff = c * N
        poff = c * E

        B = [(src_v0, dst_v0, xr0, pr0, semL0, semG0),
             (src_v1, dst_v1, xr1, pr1, semL1, semG1)]

        zv = jnp.zeros((16,), jnp.float32)

        def fill_rows(ref, val):
            def body(e, carry):
                for j in range(DH // 16):
                    ref[e, pl.ds(j * 16, 16)] = val
                return carry
            lax.fori_loop(0, CH, body, 0)

        def zero_acc_span():
            def span(nspan):
                for j in range(nspan):
                    sl = pl.ds(s * ROWS_A + j * CH, CH)
                    pltpu.sync_copy(xr0, acc.at[sl])

            @pl.when(s < NTILES - 1)
            def _():
                span(ROWS_A // CH)

            @pl.when(s == NTILES - 1)
            def _():
                span(ROWS_B // CH)

        def issue_loads(ch, b):
            src_v, dst_v, _, p_rows, semL, _ = B[b]
            base = s * EPT + ch * CH
            pltpu.async_copy(src_hbm.at[pl.ds(poff + base, CH)], src_v, semL)
            pltpu.async_copy(dst_hbm.at[pl.ds(base, CH)], dst_v, semL)
            pltpu.async_copy(p_cat.at[pl.ds(poff + base, CH)], p_rows, semL)

        def drain_loads(b):
            src_v, dst_v, _, p_rows, semL, _ = B[b]
            pltpu.make_async_copy(src_hbm.at[pl.ds(0, CH)], src_v, semL).wait()
            pltpu.make_async_copy(dst_hbm.at[pl.ds(0, CH)], dst_v, semL).wait()
            pltpu.make_async_copy(p_cat.at[pl.ds(0, CH)], p_rows, semL).wait()

        def issue_gather(b):
            src_v, _, x_rows, _, _, semG = B[b]
            pltpu.async_copy(x_cat.at[src_v], x_rows, semG)

        def drain_gather(b):
            _, _, x_rows, _, _, semG = B[b]
            pltpu.make_async_copy(x_cat.at[pl.ds(0, CH)], x_rows, semG).wait()

        def mul_scatter(b):
            _, dst_v, x_rows, p_rows, _, _ = B[b]

            def mrow(e, inner):
                for j in range(DH // 16):
                    sl = pl.ds(j * 16, 16)
                    x_rows[e, sl] = x_rows[e, sl] * p_rows[e, sl]
                return inner

            lax.fori_loop(0, CH, mrow, 0)
            pltpu.sync_copy(x_rows, acc.at[dst_v], add=True)

        fill_rows(xr0, zv)
        zero_acc_span()
        plsc.subcore_barrier()

        issue_loads(0, 0)

        def pair(m, carry):
            for b in (0, 1):
                ch = 2 * m + b

                @pl.when(ch < NCHUNK)
                def _():
                    drain_loads(b)
                    issue_gather(b)

                @pl.when(ch >= 1)
                def _():
                    drain_gather(1 - b)
                    mul_scatter(1 - b)

                @pl.when(ch < NCHUNK - 1)
                def _():
                    issue_loads(ch + 1, 1 - b)
            return carry

        lax.fori_loop(0, (NCHUNK + 2) // 2, pair, 0)

        plsc.subcore_barrier()

        def dump_agg(nspan):
            for j in range(nspan):
                sl = pl.ds(s * ROWS_A + j * CH, CH)
                osl = pl.ds(xoff + s * ROWS_A + j * CH, CH)
                pltpu.sync_copy(acc.at[sl], xr0)
                pltpu.sync_copy(xr0, out_cat.at[osl])

        @pl.when(s < NTILES - 1)
        def _():
            dump_agg(ROWS_A // CH)

        @pl.when(s == NTILES - 1)
        def _():
            dump_agg(ROWS_B // CH)

        plsc.subcore_barrier()

        fill_rows(xr0, zv)
        zero_acc_span()
        ov = jnp.full((16,), 1.0, jnp.float32)
        fill_rows(pr0, ov)
        plsc.subcore_barrier()

        def issue_dst(ch, b):
            dst_v, semL = B[b][1], B[b][4]
            pltpu.async_copy(dst_hbm.at[pl.ds(s * EPT + ch * CH, CH)],
                             dst_v, semL)

        def drain_dst(b):
            dst_v, semL = B[b][1], B[b][4]
            pltpu.make_async_copy(dst_hbm.at[pl.ds(0, CH)], dst_v, semL).wait()

        issue_dst(0, 0)

        def pair_deg(m, carry):
            for b in (0, 1):
                ch = 2 * m + b

                @pl.when(ch < NCHUNK)
                def _():
                    drain_dst(b)

                @pl.when(ch < NCHUNK - 1)
                def _():
                    issue_dst(ch + 1, 1 - b)

                @pl.when(ch < NCHUNK)
                def _():
                    pltpu.sync_copy(pr0, acc.at[B[b][1]], add=True)
            return carry

        lax.fori_loop(0, (NCHUNK + 1) // 2, pair_deg, 0)

        plsc.subcore_barrier()

        HN = N // 2
        DR_A = 320
        DR_B = HN - 15 * DR_A

        def dump_deg(spans):
            for (off, ln) in spans:
                pltpu.sync_copy(acc.at[pl.ds(c * HN + off, ln)],
                                xr0.at[pl.ds(0, ln)])
                pltpu.sync_copy(xr0.at[pl.ds(0, ln)],
                                deg_out.at[pl.ds(c * HN + off, ln)])

        @pl.when(s < NTILES - 1)
        def _():
            dump_deg([(s * DR_A + j * CH, CH) for j in range(DR_A // CH)])

        @pl.when(s == NTILES - 1)
        def _():
            dump_deg([(15 * DR_A, CH), (15 * DR_A + CH, CH),
                      (15 * DR_A + 2 * CH, DR_B - 2 * CH)])

    return _sc_agg


_NBLK = 1000


def _fin_body(x_ref, lo_ref, hi_ref, deg_ref, g_ref, b_ref, out_ref):
    x = x_ref[...]
    agg = jnp.concatenate([lo_ref[...], hi_ref[...]], axis=1)
    deg = jnp.maximum(deg_ref[:, 0:1], 1.0)
    h = x + 0.5 * agg / deg
    mu = jnp.mean(h, axis=1, keepdims=True)
    xc = h - mu
    var = jnp.mean(xc * xc, axis=1, keepdims=True)
    out_ref[...] = xc * lax.rsqrt(var + 1e-5) * g_ref[...] + b_ref[...]


def _finalize(x, agg_lo, agg_hi, deg, g, b):
    grid = (N // _NBLK,)
    return pl.pallas_call(
        _fin_body,
        grid=grid,
        in_specs=[
            pl.BlockSpec((_NBLK, D), lambda i: (i, 0)),
            pl.BlockSpec((_NBLK, DH), lambda i: (i, 0)),
            pl.BlockSpec((_NBLK, DH), lambda i: (N // _NBLK + i, 0)),
            pl.BlockSpec((_NBLK, 128), lambda i: (i, 0)),
            pl.BlockSpec((1, D), lambda i: (0, 0)),
            pl.BlockSpec((1, D), lambda i: (0, 0)),
        ],
        out_specs=pl.BlockSpec((_NBLK, D), lambda i: (i, 0)),
        out_shape=jax.ShapeDtypeStruct((N, D), jnp.float32),
    )(x, agg_lo, agg_hi, deg, g.reshape(1, D), b.reshape(1, D))


def kernel(x_user, x_item, edge_index_ui, edge_index_iu, edge_feat_ui,
           edge_feat_iu, p_ui, p_iu, ln_e_g, ln_e_b, W1, b1, W2, b2,
           ln_u_g, ln_u_b, ln_i_g, ln_i_b):
    src_ui, dst_ui = edge_index_ui[0], edge_index_ui[1]
    src_iu, dst_iu = edge_index_iu[0], edge_index_iu[1]

    p_ui_cat = _edge_prompt(edge_feat_ui, ln_e_g, ln_e_b, W1, b1, W2,
                            b2, p_ui).reshape(2 * E, DH)
    p_iu_cat = _edge_prompt(edge_feat_iu, ln_e_g, ln_e_b, W1, b1, W2,
                            b2, p_iu).reshape(2 * E, DH)

    xu_cat = jnp.concatenate([x_user[:, :DH], x_user[:, DH:]], axis=0)
    xi_cat = jnp.concatenate([x_item[:, :DH], x_item[:, DH:]], axis=0)

    src_ui2 = jnp.concatenate([src_ui, src_ui + N])
    src_iu2 = jnp.concatenate([src_iu, src_iu + N])

    sc_agg = _make_sc_agg()
    agg_i, deg_i = sc_agg(xu_cat, src_ui2, dst_ui, p_ui_cat)
    agg_u, deg_u = sc_agg(xi_cat, src_iu2, dst_iu, p_iu_cat)

    out_user = _finalize(x_user, agg_u, agg_u, deg_u, ln_u_g, ln_u_b)
    out_item = _finalize(x_item, agg_i, agg_i, deg_i, ln_i_g, ln_i_b)
    return (out_user, out_item)
